# Initial kernel scaffold; baseline (speedup 1.0000x reference)
import jax
import jax.numpy as jnp
from jax.experimental import pallas as pl


def kernel(x1, x2, neg_indices):
    raise NotImplementedError("write your pallas kernel here")


# speedup vs baseline: 10.8847x; 10.8847x over previous
import functools

import jax
import jax.numpy as jnp
from jax import lax
from jax.experimental import pallas as pl
from jax.experimental.pallas import tpu as pltpu
from jax.experimental.pallas import tpu_sc as plsc

N = 4096
D = 96
K = 100
L = 16
NC = 2
NS = 16
NW = NC * NS
RPW = N // NW
NB = (K + L - 1) // L
DC = D // L


def _prep_body(x1_ref, x2_ref, x1n_ref, pos_ref):
    x1 = x1_ref[...]
    x2 = x2_ref[...]
    n1 = jnp.sqrt(jnp.sum(x1 * x1, axis=1, keepdims=True))
    n2 = jnp.sqrt(jnp.sum(x2 * x2, axis=1, keepdims=True))
    x1n = x1 / jnp.maximum(n1, 1e-12)
    x2n = x2 / jnp.maximum(n2, 1e-12)
    x1n_ref[...] = x1n
    pos_ref[...] = jnp.sum(jnp.exp(x1n * x2n), axis=1)


_prep = pl.pallas_call(
    _prep_body,
    out_shape=[
        jax.ShapeDtypeStruct((N, D), jnp.float32),
        jax.ShapeDtypeStruct((N,), jnp.float32),
    ],
)


def _loss_body(pos_ref, neg_ref, out_ref):
    p = pos_ref[...]
    n = neg_ref[...]
    out_ref[0] = jnp.mean(jnp.log(p + n) - jnp.log(p))


_loss = pl.pallas_call(
    _loss_body,
    out_specs=pl.BlockSpec(memory_space=pltpu.SMEM),
    out_shape=jax.ShapeDtypeStruct((1,), jnp.float32),
)


def _compute_row(x1v, gbuf, i):
    xr = [x1v[i, pl.ds(c * L, L)] for c in range(DC)]
    lanes = lax.iota(jnp.int32, L)
    total = jnp.zeros((L,), jnp.float32)
    for b in range(NB):
        nk = min(L, K - b * L)
        tot = jnp.zeros((L,), jnp.float32)
        for j in range(nk):
            k = b * L + j
            acc = xr[0] * gbuf[k, pl.ds(0, L)]
            for c in range(1, DC):
                acc = acc + xr[c] * gbuf[k, pl.ds(c * L, L)]
            tot = jnp.where(lanes == j, jnp.sum(acc), tot)
        ex = jnp.exp(tot)
        if nk < L:
            ex = jnp.where(lanes < nk, ex, 0.0)
        total = total + ex
    return jnp.sum(total)


def _neg_body(x1n_hbm, idx_hbm, out_hbm, x1v, idxv, g0, g1, outv,
              s0, s1):
    wid = lax.axis_index("s") * NC + lax.axis_index("c")
    base_row = wid * RPW
    pltpu.sync_copy(x1n_hbm.at[pl.ds(base_row, RPW)], x1v)
    pltpu.sync_copy(idx_hbm.at[pl.ds(base_row, RPW)], idxv)

    pltpu.async_copy(x1n_hbm.at[idxv.at[0]], g0, s0)
    lanes = lax.iota(jnp.int32, L)

    def body(ii, rowvec):
        i0 = 2 * ii
        i1 = i0 + 1
        pltpu.async_copy(x1n_hbm.at[idxv.at[i1]], g1, s1)
        pltpu.make_async_copy(x1n_hbm.at[idxv.at[i0]], g0, s0).wait()
        v0 = _compute_row(x1v, g0, i0)
        rowvec = jnp.where(lanes == jnp.bitwise_and(i0, L - 1), v0, rowvec)

        @pl.when(i1 + 1 < RPW)
        def _():
            pltpu.async_copy(x1n_hbm.at[idxv.at[i1 + 1]], g0, s0)

        pltpu.make_async_copy(x1n_hbm.at[idxv.at[i1]], g1, s1).wait()
        v1 = _compute_row(x1v, g1, i1)
        rowvec = jnp.where(lanes == jnp.bitwise_and(i1, L - 1), v1, rowvec)

        @pl.when(jnp.bitwise_and(i1, L - 1) == L - 1)
        def _():
            outv[pl.ds(i1 - (L - 1), L)] = rowvec

        return rowvec

    lax.fori_loop(0, RPW // 2, body, jnp.zeros((L,), jnp.float32))
    pltpu.sync_copy(outv, out_hbm.at[pl.ds(base_row, RPW)])


_neg = functools.partial(
    pl.kernel,
    out_type=jax.ShapeDtypeStruct((N,), jnp.float32),
    mesh=plsc.VectorSubcoreMesh(core_axis_name="c", subcore_axis_name="s"),
    compiler_params=pltpu.CompilerParams(
        needs_layout_passes=False, use_tc_tiling_on_sc=False
    ),
    scratch_types=[
        pltpu.VMEM((RPW, D), jnp.float32),
        pltpu.VMEM((RPW, K), jnp.int32),
        pltpu.VMEM((K, D), jnp.float32),
        pltpu.VMEM((K, D), jnp.float32),
        pltpu.VMEM((RPW,), jnp.float32),
        pltpu.SemaphoreType.DMA,
        pltpu.SemaphoreType.DMA,
    ],
)(_neg_body)


def kernel(x1, x2, neg_indices):
    b, d, h, w = x1.shape
    flat_x1 = jnp.transpose(x1, (0, 2, 3, 1)).reshape(-1, d)
    flat_x2 = jnp.transpose(x2, (0, 2, 3, 1)).reshape(-1, d)
    idx = neg_indices.astype(jnp.int32)
    x1n, pos = _prep(flat_x1, flat_x2)
    neg = _neg(x1n, idx)
    loss = _loss(pos, neg)
    return loss.reshape(())


# speedup vs baseline: 11.2945x; 1.0376x over previous
import functools

import jax
import jax.numpy as jnp
from jax import lax
from jax.experimental import pallas as pl
from jax.experimental.pallas import tpu as pltpu
from jax.experimental.pallas import tpu_sc as plsc

N = 4096
D = 96
K = 100
L = 16
NC = 2
NS = 16
NW = NC * NS
RPW = N // NW
NB = (K + L - 1) // L
DC = D // L


def _prep_body(x1_ref, x2_ref, tbl_ref, pos_ref):
    x1 = x1_ref[...]
    x2 = x2_ref[...]
    n1 = jnp.sqrt(jnp.sum(x1 * x1, axis=1, keepdims=True))
    n2 = jnp.sqrt(jnp.sum(x2 * x2, axis=1, keepdims=True))
    x1n = x1 / jnp.maximum(n1, 1e-12)
    x2n = x2 / jnp.maximum(n2, 1e-12)
    tbl_ref[...] = x1n.astype(jnp.bfloat16)
    pos_ref[...] = jnp.sum(jnp.exp(x1n * x2n), axis=1)


_prep = pl.pallas_call(
    _prep_body,
    out_shape=[
        jax.ShapeDtypeStruct((N, D), jnp.bfloat16),
        jax.ShapeDtypeStruct((N,), jnp.float32),
    ],
)


def _loss_body(pos_ref, neg_ref, out_ref):
    p = pos_ref[...]
    n = neg_ref[...]
    out_ref[0] = jnp.mean(jnp.log(p + n) - jnp.log(p))


_loss = pl.pallas_call(
    _loss_body,
    out_specs=pl.BlockSpec(memory_space=pltpu.SMEM),
    out_shape=jax.ShapeDtypeStruct((1,), jnp.float32),
)


DC2 = D // (2 * L)


def _unpacked_row(ref, i):
    out = []
    for c in range(DC2):
        ab = ref[i, pl.ds(c * 2 * L, 2 * L)]
        a, b = plsc.unpack(ab, format=plsc.PackFormat.INTERLEAVED)
        out.append(a)
        out.append(b)
    return out


def _compute_row(x1v, gbuf, tbuf, i):
    xr = _unpacked_row(x1v, i)
    lanes = lax.iota(jnp.int32, L)
    total = jnp.zeros((L,), jnp.float32)
    for b in range(NB):
        nk = min(L, K - b * L)
        for j in range(nk):
            g = _unpacked_row(gbuf, b * L + j)
            acc = xr[0] * g[0]
            for c in range(1, 2 * DC2):
                acc = acc + xr[c] * g[c]
            tbuf[j, :] = acc
        cols = [
            plsc.load_gather(tbuf, [lanes, jnp.full((L,), l, jnp.int32)])
            for l in range(L)
        ]
        while len(cols) > 1:
            cols = [cols[p] + cols[p + 1] for p in range(0, len(cols), 2)]
        ex = jnp.exp(cols[0])
        if nk < L:
            ex = jnp.where(lanes < nk, ex, 0.0)
        total = total + ex
    return jnp.sum(total)


def _neg_body(x1n_hbm, idx_hbm, out_hbm, x1v, idxv, g0, g1, outv, tbuf,
              s0, s1):
    wid = lax.axis_index("s") * NC + lax.axis_index("c")
    base_row = wid * RPW
    pltpu.sync_copy(x1n_hbm.at[pl.ds(base_row, RPW)], x1v)
    pltpu.sync_copy(idx_hbm.at[pl.ds(base_row, RPW)], idxv)

    pltpu.async_copy(x1n_hbm.at[idxv.at[0]], g0, s0)
    lanes = lax.iota(jnp.int32, L)

    def body(ii, rowvec):
        i0 = 2 * ii
        i1 = i0 + 1
        pltpu.async_copy(x1n_hbm.at[idxv.at[i1]], g1, s1)
        pltpu.make_async_copy(x1n_hbm.at[idxv.at[i0]], g0, s0).wait()
        v0 = _compute_row(x1v, g0, tbuf, i0)
        rowvec = jnp.where(lanes == jnp.bitwise_and(i0, L - 1), v0, rowvec)

        @pl.when(i1 + 1 < RPW)
        def _():
            pltpu.async_copy(x1n_hbm.at[idxv.at[i1 + 1]], g0, s0)

        pltpu.make_async_copy(x1n_hbm.at[idxv.at[i1]], g1, s1).wait()
        v1 = _compute_row(x1v, g1, tbuf, i1)
        rowvec = jnp.where(lanes == jnp.bitwise_and(i1, L - 1), v1, rowvec)

        @pl.when(jnp.bitwise_and(i1, L - 1) == L - 1)
        def _():
            outv[pl.ds(i1 - (L - 1), L)] = rowvec

        return rowvec

    lax.fori_loop(0, RPW // 2, body, jnp.zeros((L,), jnp.float32))
    pltpu.sync_copy(outv, out_hbm.at[pl.ds(base_row, RPW)])


_neg = functools.partial(
    pl.kernel,
    out_type=jax.ShapeDtypeStruct((N,), jnp.float32),
    mesh=plsc.VectorSubcoreMesh(core_axis_name="c", subcore_axis_name="s"),
    compiler_params=pltpu.CompilerParams(
        needs_layout_passes=False, use_tc_tiling_on_sc=False
    ),
    scratch_types=[
        pltpu.VMEM((RPW, D), jnp.bfloat16),
        pltpu.VMEM((RPW, K), jnp.int32),
        pltpu.VMEM((K, D), jnp.bfloat16),
        pltpu.VMEM((K, D), jnp.bfloat16),
        pltpu.VMEM((RPW,), jnp.float32),
        pltpu.VMEM((L, L), jnp.float32),
        pltpu.SemaphoreType.DMA,
        pltpu.SemaphoreType.DMA,
    ],
)(_neg_body)


def kernel(x1, x2, neg_indices):
    b, d, h, w = x1.shape
    flat_x1 = jnp.transpose(x1, (0, 2, 3, 1)).reshape(-1, d)
    flat_x2 = jnp.transpose(x2, (0, 2, 3, 1)).reshape(-1, d)
    idx = neg_indices.astype(jnp.int32)
    x1n, pos = _prep(flat_x1, flat_x2)
    neg = _neg(x1n, idx)
    loss = _loss(pos, neg)
    return loss.reshape(())
